# Initial kernel scaffold; baseline (speedup 1.0000x reference)
import jax
import jax.numpy as jnp
from jax.experimental import pallas as pl


def kernel(target):
    raise NotImplementedError("write your pallas kernel here")


# speedup vs baseline: 61.0983x; 61.0983x over previous
import math

import jax
import jax.numpy as jnp
import numpy as np
from jax.experimental import pallas as pl

NLAT = 128
NLON = 256
LMAX = 50
MMAX = 50
B = 4
N = 4096
G = NLAT * NLON

_PI = math.pi
_IDX_MASK = (1 << 12) - 1
_INT_MAX = np.int32(0x7FFFFFFF)


def _cc_quad(n):
    nn = n - 1
    theta = np.pi * np.arange(n) / nn
    j = np.arange(1, nn // 2 + 1)
    b = np.where(2 * j == nn, 1.0, 2.0)
    s = (b / (4.0 * j * j - 1.0))[None, :] * np.cos(2.0 * np.outer(theta, j))
    c = np.where((np.arange(n) == 0) | (np.arange(n) == nn), 1.0, 2.0)
    return (c / nn) * (1.0 - s.sum(axis=1))


def _legendre(mmax, lmax, x):
    nmax = max(mmax, lmax)
    vdm = np.zeros((nmax, nmax, len(x)))
    vdm[0, 0, :] = 1.0 / np.sqrt(4.0 * np.pi)
    for l in range(1, nmax):
        vdm[l - 1, l, :] = np.sqrt(2 * l + 1) * x * vdm[l - 1, l - 1, :]
        vdm[l, l, :] = np.sqrt((2 * l + 1) * (1 + x) * (1 - x) / (2 * l)) * vdm[l - 1, l - 1, :]
    for l in range(2, nmax):
        for m in range(0, l - 1):
            a1 = np.sqrt((2 * l - 1) / (l - m) * (2 * l + 1) / (l + m))
            a2 = np.sqrt((l + m - 1) / (l - m) * (2 * l + 1) / (2 * l - 3) * (l - m - 1) / (l + m))
            vdm[m, l, :] = x * a1 * vdm[m, l - 1, :] - a2 * vdm[m, l - 2, :]
    vdm = vdm[:mmax, :lmax].copy()
    for m in range(1, mmax, 2):
        vdm[m] *= -1.0
    return vdm


def _np_consts():
    cost = np.cos(np.pi * np.arange(NLAT) / (NLAT - 1))
    w = _cc_quad(NLAT)
    pct = _legendre(MMAX, LMAX, cost)
    w_mlk = (pct * w[None, None, :]).astype(np.float32)
    w_lmk = np.ascontiguousarray(np.transpose(w_mlk, (1, 0, 2)))
    jj = np.arange(NLON)
    m = np.arange(MMAX)
    cosm = np.cos(2.0 * np.pi * np.outer(jj, m) / NLON).astype(np.float32)
    return w_lmk, cosm


_W_LMK, _COSM = _np_consts()


def _sph_body(tx_ref, ty_ref, tz_ref, phi_ref, th_ref, rho_ref):
    x = tx_ref[...]
    y = ty_ref[...]
    z = tz_ref[...]
    rho = jnp.sqrt(x * x + y * y + z * z)
    phi_ref[...] = jnp.arctan2(y, x)
    zc = jnp.clip(z / rho, -1.0, 1.0)
    th_ref[...] = jnp.arctan2(jnp.sqrt(jnp.maximum(1.0 - zc * zc, 0.0)), zc) - _PI
    rho_ref[...] = rho


def _knn_body(s0_ref, s1_ref, ft_ref, x_ref):
    j = pl.program_id(1)
    s0 = s0_ref[...]
    s1 = s1_ref[...]
    ft = ft_ref[...]
    g0 = jax.lax.convert_element_type(j, jnp.float32) * (_PI / NLAT)
    g1i = jax.lax.broadcasted_iota(jnp.int32, (1, NLON), 1) - NLAT
    g1 = g1i.astype(jnp.float32) * (_PI / NLAT)
    c0 = s0 - g0
    c0 = c0 * c0
    t1 = s1 - g1
    d2 = c0 + t1 * t1
    bits = jax.lax.bitcast_convert_type(d2, jnp.int32)
    n_iota = jax.lax.broadcasted_iota(jnp.int32, (N, NLON), 0)
    p = (bits & jnp.int32(~_IDX_MASK)) | n_iota
    num = jnp.zeros((1, NLON), jnp.float32)
    den = jnp.zeros((1, NLON), jnp.float32)
    for _ in range(3):
        m = jnp.min(p, axis=0, keepdims=True)
        sel = p == m
        fk = jnp.sum(jnp.where(sel, ft, 0.0), axis=0, keepdims=True)
        d2k = jax.lax.bitcast_convert_type(m & jnp.int32(~_IDX_MASK), jnp.float32)
        dk = jnp.sqrt(d2k)
        num = num + fk * dk
        den = den + dk
        p = jnp.where(sel, _INT_MAX, p)
    x_ref[0, 0, 0, :] = (num / den)[0]


def _sht_body(x_ref, cosm_ref, w_ref, o_ref):
    xb = x_ref[0]
    cosm = cosm_ref[...]
    xf_t = jax.lax.dot_general(cosm, xb, (((0,), (1,)), ((), ())),
                               preferred_element_type=jnp.float32)
    xf_t = xf_t * (2.0 * _PI / NLON)
    o_ref[0] = jnp.sum(w_ref[...] * xf_t[None, :, :], axis=2)


def kernel(target):
    tx = target[:, :, 0]
    ty = target[:, :, 1]
    tz = target[:, :, 2]
    shp = jax.ShapeDtypeStruct((B, N), jnp.float32)
    phi, th, rho = pl.pallas_call(
        _sph_body,
        out_shape=[shp, shp, shp],
    )(tx, ty, tz)
    s0 = phi.reshape(B * N, 1)
    s1 = th.reshape(B * N, 1)
    ft = rho.reshape(B * N, 1)
    col = pl.BlockSpec((N, 1), lambda b, j: (b, 0))
    x = pl.pallas_call(
        _knn_body,
        grid=(B, NLAT),
        in_specs=[col, col, col],
        out_specs=pl.BlockSpec((1, 1, 1, NLON), lambda b, j: (b, j, 0, 0)),
        out_shape=jax.ShapeDtypeStruct((B, NLAT, 1, NLON), jnp.float32),
    )(s0, s1, ft)
    x = x.reshape(B, NLAT, NLON)
    coeffs = pl.pallas_call(
        _sht_body,
        grid=(B,),
        in_specs=[
            pl.BlockSpec((1, NLAT, NLON), lambda b: (b, 0, 0)),
            pl.BlockSpec((NLON, MMAX), lambda b: (0, 0)),
            pl.BlockSpec((LMAX, MMAX, NLAT), lambda b: (0, 0, 0)),
        ],
        out_specs=pl.BlockSpec((1, LMAX, MMAX), lambda b: (b, 0, 0)),
        out_shape=jax.ShapeDtypeStruct((B, LMAX, MMAX), jnp.float32),
    )(x, jnp.asarray(_COSM), jnp.asarray(_W_LMK))
    return coeffs
